# baseline (speedup 1.0000x reference)
import dataclasses
import functools

import jax
import jax.numpy as jnp
from jax import lax
from jax.experimental import pallas as pl
from jax.experimental.pallas import tpu as pltpu
from jax.experimental.pallas import tpu_sc as plsc

NC = 2
NS = 16
L = 16
NW = NC * NS

ROWS, COLS = 4096, 200
CPW = ROWS // NW


@jax.jit
def _sc_lookup_t(inputs_t, mapping):
    mesh = plsc.VectorSubcoreMesh(
        core_axis_name="c", subcore_axis_name="s",
        num_cores=NC, num_subcores=NS)
    cp = pltpu.CompilerParams()
    if "needs_layout_passes" in pltpu.CompilerParams.__dataclass_fields__:
        cp = dataclasses.replace(cp, needs_layout_passes=False,
                                 use_tc_tiling_on_sc=True)

    @functools.partial(
        pl.kernel,
        out_type=jax.ShapeDtypeStruct((COLS, ROWS), jnp.int32),
        mesh=mesh,
        scratch_types=[
            pltpu.VMEM((128,), jnp.int32),
            pltpu.VMEM((COLS, CPW), jnp.int32),
            pltpu.VMEM((COLS, CPW), jnp.int32),
            pltpu.SemaphoreType.DMA,
            pltpu.SemaphoreType.DMA,
            pltpu.SemaphoreType.DMA,
            pltpu.SemaphoreType.DMA,
            pltpu.SemaphoreType.DMA,
        ],
        compiler_params=cp,
    )
    def lookup_kernel(in_hbm, map_hbm, out_hbm, table_v, idx_v, out_v,
                      si0, si1, si2, si3, so):
        wid = lax.axis_index("s") * NC + lax.axis_index("c")
        col0 = wid * CPW
        bands = ((0, 56, si0), (56, 56, si1), (112, 56, si2), (168, 32, si3))
        cins = [
            pltpu.async_copy(
                in_hbm.at[pl.ds(r0, nr), pl.ds(col0, CPW)],
                idx_v.at[pl.ds(r0, nr)], sem)
            for r0, nr, sem in bands
        ]
        pltpu.sync_copy(map_hbm, table_v)
        couts = []
        for (r0, nr, _), cin in zip(bands, cins):
            cin.wait()

            @plsc.parallel_loop(r0, r0 + nr, step=1, unroll=4)
            def _(r):
                for o in range(0, CPW, L):
                    idx = idx_v[r, pl.ds(o, L)]
                    out_v[r, pl.ds(o, L)] = plsc.load_gather(table_v, [idx])

            couts.append(pltpu.async_copy(
                out_v.at[pl.ds(r0, nr)],
                out_hbm.at[pl.ds(r0, nr), pl.ds(col0, CPW)], so))
        for cout in couts:
            cout.wait()

    return lookup_kernel(inputs_t, mapping)


def kernel(inputs, mapping):
    return _sc_lookup_t(inputs.T, mapping).T
